# baseline (speedup 1.0000x reference)
import jax
import jax.numpy as jnp
from jax import lax
from jax.experimental import pallas as pl
from jax.experimental.pallas import tpu as pltpu
from jax.experimental.pallas import tpu_sc as plsc

N_NODES = 100000
EMB = 128
NSEG = 512
NTASK = 10
NC = 2
NS = 16
NW = NC * NS
SZ = 160
HSZ = 80
NCH = N_NODES // SZ
GROUPS = SZ // 16
JFULL = NCH // NW
JREM = NCH - JFULL * NW


def _sc_body(x_hbm, batch_hbm, sum_hbm, cnt_hbm, xbuf0, xbuf1, bbuf0, bbuf1,
             cacc, zbuf, sacc, sem0, sem1, ssem0, ssem1):
    c = lax.axis_index("c")
    s = lax.axis_index("s")
    w = c * NS + s
    xbufs = (xbuf0, xbuf1)
    bbufs = (bbuf0, bbuf1)
    sems = (sem0, sem1)
    ssems = (ssem0, ssem1)

    zero = jnp.zeros((16,), jnp.float32)
    ones = jnp.ones((16,), jnp.float32)

    def zc(i, carry):
        for k in range(8):
            cacc[i, pl.ds(k * 16, 16)] = zero
        return carry

    lax.fori_loop(0, 64, zc, 0)

    def zz(i, carry):
        for k in range(8):
            zbuf[i, pl.ds(k * 16, 16)] = zero
        return carry

    lax.fori_loop(0, 64, zz, 0)

    @pl.when(s == 0)
    def _():
        for blk in range(8):
            pltpu.sync_copy(zbuf, sacc.at[pl.ds(blk * 64, 64)])

    plsc.subcore_barrier()

    myn = jnp.where(w < JREM, JFULL + 1, JFULL)

    def start_load(j, b):
        base = (w + NW * j) * SZ
        pltpu.async_copy(x_hbm.at[pl.ds(base, SZ)], xbufs[b], sems[b])
        pltpu.async_copy(batch_hbm.at[pl.ds(base, HSZ)], bbufs[b].at[0], sems[b])
        pltpu.async_copy(batch_hbm.at[pl.ds(base + HSZ, HSZ)], bbufs[b].at[1],
                         sems[b])

    def wait_load(b):
        pltpu.make_async_copy(x_hbm.at[pl.ds(0, SZ)], xbufs[b], sems[b]).wait()
        pltpu.make_async_copy(batch_hbm.at[pl.ds(0, HSZ)], bbufs[b].at[0],
                              sems[b]).wait()
        pltpu.make_async_copy(batch_hbm.at[pl.ds(0, HSZ)], bbufs[b].at[1],
                              sems[b]).wait()

    def wait_scatter(b):
        for h in range(2):
            pltpu.make_async_copy(xbufs[b].at[pl.ds(h * HSZ, HSZ)],
                                  sacc.at[bbufs[b].at[h]], ssems[b]).wait()

    def handle(j, b):
        @pl.when(j < myn)
        def _():
            wait_load(b)

            @pl.when(j + 1 < myn)
            def _():
                @pl.when(j >= 1)
                def _():
                    wait_scatter(1 - b)

                start_load(j + 1, 1 - b)

            for h in range(2):
                pltpu.async_copy(xbufs[b].at[pl.ds(h * HSZ, HSZ)],
                                 sacc.at[bbufs[b].at[h]], ssems[b], add=True)

            def gb(g, carry2):
                segs = bbufs[b][g // 5, pl.ds((g % 5) * 16, 16)]
                for k in range(16):
                    seg = segs[k]
                    srow = seg // 8
                    scol = (seg % 8) * 16
                    plsc.addupdate(cacc.at[srow, pl.ds(scol, 16)], ones)
                return carry2

            lax.fori_loop(0, GROUPS, gb, 0)

    @pl.when(myn > 0)
    def _():
        start_load(0, 0)

    def jb(jj, carry):
        handle(2 * jj, 0)
        handle(2 * jj + 1, 1)
        return carry

    lax.fori_loop(0, (JFULL + 2) // 2, jb, 0)

    wait_scatter(0)
    wait_scatter(1)

    plsc.subcore_barrier()

    @pl.when(s == 0)
    def _():
        pltpu.sync_copy(sacc, sum_hbm.at[c])

    pltpu.sync_copy(cacc, cnt_hbm.at[w])


def _segment_partials(x, batch32):
    mesh = plsc.VectorSubcoreMesh(core_axis_name="c", subcore_axis_name="s")
    f = pl.kernel(
        _sc_body,
        mesh=mesh,
        out_type=(
            jax.ShapeDtypeStruct((NC, NSEG, EMB), jnp.float32),
            jax.ShapeDtypeStruct((NW, 64, 128), jnp.float32),
        ),
        scratch_types=[
            pltpu.VMEM((SZ, EMB), jnp.float32),
            pltpu.VMEM((SZ, EMB), jnp.float32),
            pltpu.VMEM((2, HSZ), jnp.int32),
            pltpu.VMEM((2, HSZ), jnp.int32),
            pltpu.VMEM((64, 128), jnp.float32),
            pltpu.VMEM((64, 128), jnp.float32),
            pltpu.VMEM_SHARED((NSEG, EMB), jnp.float32),
            pltpu.SemaphoreType.DMA,
            pltpu.SemaphoreType.DMA,
            pltpu.SemaphoreType.DMA,
            pltpu.SemaphoreType.DMA,
        ],
    )
    return f(x, batch32)


def _final_body(sum_ref, cnt_ref, w_ref, b_ref, o_ref):
    S = jnp.sum(sum_ref[...], axis=0)
    T = jnp.sum(cnt_ref[...], axis=0)
    si = lax.broadcasted_iota(jnp.int32, (NSEG, 64), 0)
    ri = lax.broadcasted_iota(jnp.int32, (NSEG, 64), 1)
    R2 = jnp.where(ri == si // 8, 1.0, 0.0)
    M = lax.dot_general(R2, T, (((1,), (0,)), ((), ())),
                        preferred_element_type=jnp.float32)
    li = lax.broadcasted_iota(jnp.int32, (NSEG, 128), 1)
    s2 = lax.broadcasted_iota(jnp.int32, (NSEG, 128), 0)
    msk = jnp.where(li // 16 == s2 % 8, 1.0, 0.0)
    cnt = jnp.sum(M * msk, axis=1, keepdims=True) / 16.0
    mean = S / jnp.maximum(cnt, 1.0)
    out = lax.dot_general(mean, w_ref[...], (((1,), (1,)), ((), ())),
                          preferred_element_type=jnp.float32)
    o_ref[...] = out + b_ref[...]


def _finalize(sums, cnts, W, b2):
    return pl.pallas_call(
        _final_body,
        grid=(1,),
        in_specs=[
            pl.BlockSpec((NC, NSEG, EMB), lambda i: (0, 0, 0)),
            pl.BlockSpec((NW, 64, 128), lambda i: (0, 0, 0)),
            pl.BlockSpec((NTASK, EMB), lambda i: (0, 0)),
            pl.BlockSpec((1, NTASK), lambda i: (0, 0)),
        ],
        out_specs=pl.BlockSpec((NSEG, NTASK), lambda i: (0, 0)),
        out_shape=jax.ShapeDtypeStruct((NSEG, NTASK), jnp.float32),
    )(sums, cnts, W, b2)


def kernel(x, batch, W, b):
    batch32 = batch.astype(jnp.int32)
    sums, cnts = _segment_partials(x, batch32)
    return _finalize(sums, cnts, W, b.reshape(1, NTASK))


# speedup vs baseline: 1.0091x; 1.0091x over previous
import jax
import jax.numpy as jnp
from jax import lax
from jax.experimental import pallas as pl
from jax.experimental.pallas import tpu as pltpu
from jax.experimental.pallas import tpu_sc as plsc

N_NODES = 100000
EMB = 128
NSEG = 512
NTASK = 10
NC = 2
NS = 16
NW = NC * NS
SZ = 400
HSZ = 80
NSUB = SZ // HSZ
NCH = N_NODES // SZ
GROUPS = SZ // 16
JFULL = NCH // NW
JREM = NCH - JFULL * NW


def _sc_body(x_hbm, batch_hbm, sum_hbm, cnt_hbm, xbuf0, xbuf1, bbuf0, bbuf1,
             cacc, zbuf, sacc, sem0, sem1, ssem0, ssem1):
    c = lax.axis_index("c")
    s = lax.axis_index("s")
    w = c * NS + s
    xbufs = (xbuf0, xbuf1)
    bbufs = (bbuf0, bbuf1)
    sems = (sem0, sem1)
    ssems = (ssem0, ssem1)

    zero = jnp.zeros((16,), jnp.float32)
    ones = jnp.ones((16,), jnp.float32)

    def zc(i, carry):
        for k in range(8):
            cacc[i, pl.ds(k * 16, 16)] = zero
        return carry

    lax.fori_loop(0, 64, zc, 0)

    def zz(i, carry):
        for k in range(8):
            zbuf[i, pl.ds(k * 16, 16)] = zero
        return carry

    lax.fori_loop(0, 64, zz, 0)

    @pl.when(s == 0)
    def _():
        for blk in range(8):
            pltpu.sync_copy(zbuf, sacc.at[pl.ds(blk * 64, 64)])

    plsc.subcore_barrier()

    myn = jnp.where(w < JREM, JFULL + 1, JFULL)

    def start_load(j, b):
        base = (w + NW * j) * SZ
        pltpu.async_copy(x_hbm.at[pl.ds(base, SZ)], xbufs[b], sems[b])
        for h in range(NSUB):
            pltpu.async_copy(batch_hbm.at[pl.ds(base + h * HSZ, HSZ)],
                             bbufs[b].at[h], sems[b])

    def wait_load(b):
        pltpu.make_async_copy(x_hbm.at[pl.ds(0, SZ)], xbufs[b], sems[b]).wait()
        for h in range(NSUB):
            pltpu.make_async_copy(batch_hbm.at[pl.ds(0, HSZ)], bbufs[b].at[h],
                                  sems[b]).wait()

    def wait_scatter(b):
        for h in range(NSUB):
            pltpu.make_async_copy(xbufs[b].at[pl.ds(h * HSZ, HSZ)],
                                  sacc.at[bbufs[b].at[h]], ssems[b]).wait()

    def handle(j, b):
        @pl.when(j < myn)
        def _():
            wait_load(b)

            @pl.when(j + 1 < myn)
            def _():
                @pl.when(j >= 1)
                def _():
                    wait_scatter(1 - b)

                start_load(j + 1, 1 - b)

            for h in range(NSUB):
                pltpu.async_copy(xbufs[b].at[pl.ds(h * HSZ, HSZ)],
                                 sacc.at[bbufs[b].at[h]], ssems[b], add=True)

            def gb(g, carry2):
                segs = bbufs[b][g // 5, pl.ds((g % 5) * 16, 16)]
                for k in range(16):
                    seg = segs[k]
                    srow = seg // 8
                    scol = (seg % 8) * 16
                    plsc.addupdate(cacc.at[srow, pl.ds(scol, 16)], ones)
                return carry2

            lax.fori_loop(0, GROUPS, gb, 0)

    @pl.when(myn > 0)
    def _():
        start_load(0, 0)

    def jb(jj, carry):
        handle(2 * jj, 0)
        handle(2 * jj + 1, 1)
        return carry

    lax.fori_loop(0, (JFULL + 2) // 2, jb, 0)

    wait_scatter(0)
    wait_scatter(1)

    plsc.subcore_barrier()

    @pl.when(s == 0)
    def _():
        pltpu.sync_copy(sacc, sum_hbm.at[c])

    pltpu.sync_copy(cacc, cnt_hbm.at[w])


def _segment_partials(x, batch32):
    mesh = plsc.VectorSubcoreMesh(core_axis_name="c", subcore_axis_name="s")
    f = pl.kernel(
        _sc_body,
        mesh=mesh,
        out_type=(
            jax.ShapeDtypeStruct((NC, NSEG, EMB), jnp.float32),
            jax.ShapeDtypeStruct((NW, 64, 128), jnp.float32),
        ),
        scratch_types=[
            pltpu.VMEM((SZ, EMB), jnp.float32),
            pltpu.VMEM((SZ, EMB), jnp.float32),
            pltpu.VMEM((NSUB, HSZ), jnp.int32),
            pltpu.VMEM((NSUB, HSZ), jnp.int32),
            pltpu.VMEM((64, 128), jnp.float32),
            pltpu.VMEM((64, 128), jnp.float32),
            pltpu.VMEM_SHARED((NSEG, EMB), jnp.float32),
            pltpu.SemaphoreType.DMA,
            pltpu.SemaphoreType.DMA,
            pltpu.SemaphoreType.DMA,
            pltpu.SemaphoreType.DMA,
        ],
    )
    return f(x, batch32)


def _final_body(sum_ref, cnt_ref, w_ref, b_ref, o_ref):
    S = jnp.sum(sum_ref[...], axis=0)
    T = jnp.sum(cnt_ref[...], axis=0)
    si = lax.broadcasted_iota(jnp.int32, (NSEG, 64), 0)
    ri = lax.broadcasted_iota(jnp.int32, (NSEG, 64), 1)
    R2 = jnp.where(ri == si // 8, 1.0, 0.0)
    M = lax.dot_general(R2, T, (((1,), (0,)), ((), ())),
                        preferred_element_type=jnp.float32)
    li = lax.broadcasted_iota(jnp.int32, (NSEG, 128), 1)
    s2 = lax.broadcasted_iota(jnp.int32, (NSEG, 128), 0)
    msk = jnp.where(li // 16 == s2 % 8, 1.0, 0.0)
    cnt = jnp.sum(M * msk, axis=1, keepdims=True) / 16.0
    mean = S / jnp.maximum(cnt, 1.0)
    out = lax.dot_general(mean, w_ref[...], (((1,), (1,)), ((), ())),
                          preferred_element_type=jnp.float32)
    o_ref[...] = out + b_ref[...]


def _finalize(sums, cnts, W, b2):
    return pl.pallas_call(
        _final_body,
        grid=(1,),
        in_specs=[
            pl.BlockSpec((NC, NSEG, EMB), lambda i: (0, 0, 0)),
            pl.BlockSpec((NW, 64, 128), lambda i: (0, 0, 0)),
            pl.BlockSpec((NTASK, EMB), lambda i: (0, 0)),
            pl.BlockSpec((1, NTASK), lambda i: (0, 0)),
        ],
        out_specs=pl.BlockSpec((NSEG, NTASK), lambda i: (0, 0)),
        out_shape=jax.ShapeDtypeStruct((NSEG, NTASK), jnp.float32),
    )(sums, cnts, W, b2)


def kernel(x, batch, W, b):
    batch32 = batch.astype(jnp.int32)
    sums, cnts = _segment_partials(x, batch32)
    return _finalize(sums, cnts, W, b.reshape(1, NTASK))
